# baseline (speedup 1.0000x reference)
import functools

import jax
import jax.numpy as jnp
from jax import lax
from jax.experimental import pallas as pl
from jax.experimental.pallas import tpu as pltpu
from jax.experimental.pallas import tpu_sc as plsc

N = 10000
NP = 10240
D = 128
BSEL = 4096
E = 320000

NC = 2
NS = 16
NW = NC * NS
CHUNK = 128
NCHUNK = 80
SLOW_CID = 0
CS = 40
CF = 120
STAGES_SLOW = (40,)
STAGES_FAST = (56, 64)
STG = 64
TOT_CHUNKS = NS * (CS + CF)
FAST_BASE = NS * CS
EPAD = TOT_CHUNKS * CHUNK
PADIDX = N
ACC_R = 10112
RPT = ACC_R // NS
WB = ((0, 128), (128, 128), (256, 128), (384, 128), (512, 120))

_MESH = dict(core_axis_name="c", subcore_axis_name="s")


def _tile_edges(e):
    pad = jnp.full((EPAD - E,), PADIDX, jnp.int32)
    return jnp.concatenate([e.astype(jnp.int32), pad]).reshape(TOT_CHUNKS, CHUNK)


def _deg_body(s1, d1, s2, d2, out, dacc, idxb):
    cid = lax.axis_index("c")
    sid = lax.axis_index("s")
    wid = cid * NS + sid

    def zero(i, carry):
        dacc[pl.ds(i * 16, 16)] = jnp.zeros((16,), jnp.float32)
        return carry

    lax.fori_loop(0, NP * 4 // 16, zero, 0)

    ones = jnp.full((16,), 1.0, jnp.float32)
    for a, arr in enumerate((s1, d1, s2, d2)):
        pltpu.sync_copy(arr.at[pl.ds(wid * NCHUNK, NCHUNK)], idxb)

        def chunk(i, carry):
            for j in range(CHUNK // 16):
                v = idxb[i, pl.ds(j * 16, 16)]
                plsc.addupdate_scatter(dacc, [v * 4 + a], ones)
            return carry

        lax.fori_loop(0, NCHUNK, chunk, 0, unroll=2)

    pltpu.sync_copy(dacc, out.at[wid])


def _sc_degrees(s1, d1, s2, d2):
    k = pl.kernel(
        _deg_body,
        out_type=jax.ShapeDtypeStruct((NW, NP * 4), jnp.float32),
        mesh=plsc.VectorSubcoreMesh(**_MESH),
        compiler_params=pltpu.CompilerParams(needs_layout_passes=False),
        scratch_types=[
            pltpu.VMEM((NP * 4,), jnp.float32),
            pltpu.VMEM((NCHUNK, CHUNK), jnp.int32),
        ],
    )
    return k(s1, d1, s2, d2).reshape(NW, NP, 4)


def _agg_body(h1, h2, s1, d1, s2, d2, out,
              sall, dall, g0, g1, acc, sem0, sem1, sem2, sem3):
    cid = lax.axis_index("c")
    sid = lax.axis_index("s")
    wid = cid * NS + sid
    base = sid * RPT

    for g, (hg, sg, dg) in enumerate(((h1, s1, d1), (h2, s2, d2))):
        def zrow(r, carry):
            for j in range(D // 16):
                g0[r, pl.ds(j * 16, 16)] = jnp.zeros((16,), jnp.float32)
            return carry

        lax.fori_loop(0, CHUNK, zrow, 0)
        for off, sz in WB:
            pltpu.sync_copy(g0.at[pl.ds(0, sz)], acc.at[pl.ds(base + off, sz)])
        plsc.subcore_barrier()

        def stage(sz, cbase):
            pltpu.sync_copy(sg.at[pl.ds(cbase, sz)], sall.at[pl.ds(0, sz)])
            pltpu.sync_copy(dg.at[pl.ds(cbase, sz)], dall.at[pl.ds(0, sz)])
            pltpu.async_copy(hg.at[sall.at[0]], g0, sem0)

            def chunk2(k, carry):
                i0 = 2 * k
                pltpu.async_copy(hg.at[sall.at[i0 + 1]], g1, sem1)
                pltpu.make_async_copy(hg.at[sall.at[i0]], g0, sem0).wait()
                pltpu.sync_copy(g0, acc.at[dall.at[i0]], add=True)
                nxt = jnp.minimum(i0 + 2, sz - 1)
                pltpu.async_copy(hg.at[sall.at[nxt]], g0, sem0)
                pltpu.make_async_copy(hg.at[sall.at[i0 + 1]], g1, sem1).wait()
                pltpu.sync_copy(g1, acc.at[dall.at[i0 + 1]], add=True)
                return carry

            lax.fori_loop(0, sz // 2, chunk2, 0)
            pltpu.make_async_copy(hg.at[sall.at[0]], g0, sem0).wait()

        @pl.when(cid == SLOW_CID)
        def _():
            off = 0
            for sz in STAGES_SLOW:
                stage(sz, sid * CS + off)
                off += sz

        @pl.when(cid != SLOW_CID)
        def _():
            off = 0
            for sz in STAGES_FAST:
                stage(sz, FAST_BASE + sid * CF + off)
                off += sz

        plsc.subcore_barrier()

        for off, sz in WB:
            pltpu.sync_copy(acc.at[pl.ds(base + off, sz)], g1.at[pl.ds(0, sz)])
            pltpu.sync_copy(g1.at[pl.ds(0, sz)],
                            out.at[g, cid, pl.ds(base + off, sz)])


def _sc_aggregate(h1, h2, s1, d1, s2, d2):
    k = pl.kernel(
        _agg_body,
        out_type=jax.ShapeDtypeStruct((2, NC, NP, D), jnp.float32),
        mesh=plsc.VectorSubcoreMesh(**_MESH),
        compiler_params=pltpu.CompilerParams(needs_layout_passes=False),
        scratch_types=[
            pltpu.VMEM((STG, CHUNK), jnp.int32),
            pltpu.VMEM((STG, CHUNK), jnp.int32),
            pltpu.VMEM((CHUNK, D), jnp.float32),
            pltpu.VMEM((CHUNK, D), jnp.float32),
            pltpu.VMEM_SHARED((ACC_R, D), jnp.float32),
            pltpu.SemaphoreType.DMA,
            pltpu.SemaphoreType.DMA,
            pltpu.SemaphoreType.DMA,
            pltpu.SemaphoreType.DMA,
        ],
    )
    return k(h1, h2, s1, d1, s2, d2)


_BPT = BSEL // NW


def _gather_body(c1, c2, idx2d, out, idxb, gbuf, sem):
    cid = lax.axis_index("c")
    sid = lax.axis_index("s")
    wid = cid * NS + sid
    pltpu.sync_copy(idx2d.at[wid], idxb)
    pltpu.async_copy(c1.at[idxb], gbuf, sem).wait()
    pltpu.sync_copy(gbuf, out.at[0, pl.ds(wid * _BPT, _BPT)])
    pltpu.async_copy(c2.at[idxb], gbuf, sem).wait()
    pltpu.sync_copy(gbuf, out.at[1, pl.ds(wid * _BPT, _BPT)])


def _sc_gather(code1, code2, idx2d):
    k = pl.kernel(
        _gather_body,
        out_type=jax.ShapeDtypeStruct((2, BSEL, D), jnp.float32),
        mesh=plsc.VectorSubcoreMesh(**_MESH),
        compiler_params=pltpu.CompilerParams(needs_layout_passes=False),
        scratch_types=[
            pltpu.VMEM((_BPT,), jnp.int32),
            pltpu.VMEM((_BPT, D), jnp.float32),
            pltpu.SemaphoreType.DMA,
        ],
    )
    return k(code1, code2, idx2d)


_RB = 1024


def _norm_scale_body(dref, x1ref, x2ref, n1s, n1d, n2s, n2d, h1ref, h2ref):
    deg = jnp.sum(dref[...], axis=0)
    norm = lax.rsqrt(jnp.where(deg > 0, deg, 1.0))
    n1s[...] = norm[:, 0:1]
    n1d[...] = norm[:, 1:2]
    n2s[...] = norm[:, 2:3]
    n2d[...] = norm[:, 3:4]
    h1ref[...] = x1ref[...] * norm[:, 0:1]
    h2ref[...] = x2ref[...] * norm[:, 2:3]


def _tc_norm_scale(deg_parts, x1, x2):
    nspec = pl.BlockSpec((_RB, 1), lambda i: (i, 0))
    xspec = pl.BlockSpec((_RB, D), lambda i: (i, 0))
    return pl.pallas_call(
        _norm_scale_body,
        grid=(NP // _RB,),
        in_specs=[pl.BlockSpec((NW, _RB, 4), lambda i: (0, i, 0)), xspec, xspec],
        out_specs=[nspec, nspec, nspec, nspec, xspec, xspec],
        out_shape=[jax.ShapeDtypeStruct((NP, 1), jnp.float32)] * 4
        + [jax.ShapeDtypeStruct((NP, D), jnp.float32)] * 2,
    )(deg_parts, x1, x2)


def _finish_body(pref, wref, bref, ndref, nsref, oref, *, relu):
    p = pref[0, 0] + pref[0, 1]
    y = jnp.dot(p, wref[...], preferred_element_type=jnp.float32)
    y = y * ndref[...] + bref[...]
    if relu:
        y = jnp.maximum(y, 0.0) * nsref[...]
    oref[...] = y


def _tc_finish(parts, g, W, b, nd, ns, relu):
    return pl.pallas_call(
        functools.partial(_finish_body, relu=relu),
        grid=(NP // _RB,),
        in_specs=[
            pl.BlockSpec((1, NC, _RB, D), lambda i: (g, 0, i, 0)),
            pl.BlockSpec((D, D), lambda i: (0, 0)),
            pl.BlockSpec((1, D), lambda i: (0, 0)),
            pl.BlockSpec((_RB, 1), lambda i: (i, 0)),
            pl.BlockSpec((_RB, 1), lambda i: (i, 0)),
        ],
        out_specs=pl.BlockSpec((_RB, D), lambda i: (i, 0)),
        out_shape=jax.ShapeDtypeStruct((NP, D), jnp.float32),
    )(parts, W, b.reshape(1, D), nd, ns)


def _buildc_body(cref, oref):
    c1 = cref[0]
    c2 = cref[1]
    inv1 = 1.0 / jnp.maximum(jnp.sqrt(jnp.sum(c1 * c1, axis=1, keepdims=True)), 1e-12)
    inv2 = 1.0 / jnp.maximum(jnp.sqrt(jnp.sum(c2 * c2, axis=1, keepdims=True)), 1e-12)
    a = c1 * inv1
    b = c2 * inv2
    s = 1.0 / jnp.sqrt(6.0)
    oref[...] = jnp.concatenate([a * s, b * s, (a + b) * s], axis=1)


def _tc_buildc(cpair):
    return pl.pallas_call(
        _buildc_body,
        grid=(BSEL // _RB,),
        in_specs=[pl.BlockSpec((2, _RB, D), lambda i: (0, i, 0))],
        out_specs=pl.BlockSpec((_RB, 3 * D), lambda i: (i, 0)),
        out_shape=jax.ShapeDtypeStruct((BSEL, 3 * D), jnp.float32),
    )(cpair)


def _zmm_body(aref, bref, oref):
    i = pl.program_id(0)
    j = pl.program_id(1)

    @pl.when(j < i)
    def _():
        oref[...] = jnp.zeros_like(oref)

    @pl.when(j >= i)
    def _():
        z = lax.dot_general(aref[...], bref[...], (((1,), (1,)), ((), ())),
                            preferred_element_type=jnp.float32)
        rows = i * _RB + lax.broadcasted_iota(jnp.int32, (_RB, _RB), 0)
        cols = j * _RB + lax.broadcasted_iota(jnp.int32, (_RB, _RB), 1)
        oref[...] = jnp.where(cols >= rows, z, 0.0)


def _tc_zmatmul(cmat):
    return pl.pallas_call(
        _zmm_body,
        grid=(BSEL // _RB, BSEL // _RB),
        in_specs=[
            pl.BlockSpec((_RB, 3 * D), lambda i, j: (i, 0)),
            pl.BlockSpec((_RB, 3 * D), lambda i, j: (j, 0)),
        ],
        out_specs=pl.BlockSpec((_RB, _RB), lambda i, j: (i, j)),
        out_shape=jax.ShapeDtypeStruct((BSEL, BSEL), jnp.float32),
    )(cmat, cmat)


def kernel(raw1, edge_index1, raw2, edge_index2, index,
           W11, b11, W12, b12, W21, b21, W22, b22):
    s1 = _tile_edges(edge_index1[0])
    d1 = _tile_edges(edge_index1[1])
    s2 = _tile_edges(edge_index2[0])
    d2 = _tile_edges(edge_index2[1])
    x1 = jnp.pad(raw1, ((0, NP - N), (0, 0)))
    x2 = jnp.pad(raw2, ((0, NP - N), (0, 0)))
    idx2d = index.astype(jnp.int32).reshape(NW, BSEL // NW)

    deg_parts = _sc_degrees(s1, d1, s2, d2)
    n1s, n1d, n2s, n2d, h1, h2 = _tc_norm_scale(deg_parts, x1, x2)
    p1 = _sc_aggregate(h1, h2, s1, d1, s2, d2)
    h1b = _tc_finish(p1, 0, W11, b11, n1d, n1s, relu=True)
    h2b = _tc_finish(p1, 1, W21, b21, n2d, n2s, relu=True)
    p2 = _sc_aggregate(h1b, h2b, s1, d1, s2, d2)
    code1 = _tc_finish(p2, 0, W12, b12, n1d, n1d, relu=False)
    code2 = _tc_finish(p2, 1, W22, b22, n2d, n2d, relu=False)
    cpair = _sc_gather(code1, code2, idx2d)
    cmat = _tc_buildc(cpair)
    z = _tc_zmatmul(cmat)
    return (z, 0)


# speedup vs baseline: 1.2138x; 1.2138x over previous
import functools

import jax
import jax.numpy as jnp
from jax import lax
from jax.experimental import pallas as pl
from jax.experimental.pallas import tpu as pltpu
from jax.experimental.pallas import tpu_sc as plsc

N = 10000
NP = 10240
D = 128
BSEL = 4096
E = 320000

NC = 2
NS = 16
NW = NC * NS
CHUNK = 128
NCHUNK = 80
HALF = NCHUNK // 2
EPAD = NW * NCHUNK * CHUNK
PADIDX = N
RPT = NP // NS

_MESH = dict(core_axis_name="c", subcore_axis_name="s")


def _tile_edges(e):
    pad = jnp.full((EPAD - E,), PADIDX, jnp.int32)
    return jnp.concatenate([e.astype(jnp.int32), pad]).reshape(NW, NCHUNK, CHUNK)


def _deg_body(s1, d1, s2, d2, out, dacc, idxb):
    cid = lax.axis_index("c")
    sid = lax.axis_index("s")
    wid = cid * NS + sid

    def zero(i, carry):
        dacc[pl.ds(i * 16, 16)] = jnp.zeros((16,), jnp.float32)
        return carry

    lax.fori_loop(0, NP * 4 // 16, zero, 0)

    ones = jnp.full((16,), 1.0, jnp.float32)
    for a, arr in enumerate((s1, d1, s2, d2)):
        pltpu.sync_copy(arr.at[wid], idxb)

        def chunk(i, carry):
            for j in range(CHUNK // 16):
                v = idxb[i, pl.ds(j * 16, 16)]
                plsc.addupdate_scatter(dacc, [v * 4 + a], ones)
            return carry

        lax.fori_loop(0, NCHUNK, chunk, 0, unroll=2)

    pltpu.sync_copy(dacc, out.at[wid])


def _sc_degrees(s1, d1, s2, d2):
    k = pl.kernel(
        _deg_body,
        out_type=jax.ShapeDtypeStruct((NW, NP * 4), jnp.float32),
        mesh=plsc.VectorSubcoreMesh(**_MESH),
        compiler_params=pltpu.CompilerParams(needs_layout_passes=False),
        scratch_types=[
            pltpu.VMEM((NP * 4,), jnp.float32),
            pltpu.VMEM((NCHUNK, CHUNK), jnp.int32),
        ],
    )
    return k(s1, d1, s2, d2).reshape(NW, NP, 4)


def _agg_body(h1, h2, s1, d1, s2, d2, out,
              sall, dall, g0, g1, acc, sem0, sem1, sem2, sem3):
    cid = lax.axis_index("c")
    sid = lax.axis_index("s")
    wid = cid * NS + sid
    base = sid * RPT

    for g, (hg, sg, dg) in enumerate(((h1, s1, d1), (h2, s2, d2))):
        def zrow(r, carry):
            for j in range(D // 16):
                g0[r, pl.ds(j * 16, 16)] = jnp.zeros((16,), jnp.float32)
            return carry

        lax.fori_loop(0, CHUNK, zrow, 0)
        for kk in range(RPT // CHUNK):
            pltpu.sync_copy(g0, acc.at[pl.ds(base + kk * CHUNK, CHUNK)])
        plsc.subcore_barrier()

        def gather_chunk(i, gbuf, sem):
            pltpu.async_copy(hg.at[sall.at[i, pl.ds(0, 64)]],
                             gbuf.at[pl.ds(0, 64)], sem)
            pltpu.async_copy(hg.at[sall.at[i, pl.ds(64, 64)]],
                             gbuf.at[pl.ds(64, 64)], sem)

        def wait_chunk(i, gbuf, sem):
            pltpu.make_async_copy(hg.at[sall.at[i, pl.ds(0, 64)]],
                                  gbuf.at[pl.ds(0, 64)], sem).wait()
            pltpu.make_async_copy(hg.at[sall.at[i, pl.ds(64, 64)]],
                                  gbuf.at[pl.ds(64, 64)], sem).wait()

        for h in range(NCHUNK // HALF):
            pltpu.sync_copy(sg.at[wid, pl.ds(h * HALF, HALF)], sall)
            pltpu.sync_copy(dg.at[wid, pl.ds(h * HALF, HALF)], dall)
            gather_chunk(0, g0, sem0)

            def chunk2(k, carry):
                i0 = 2 * k
                gather_chunk(i0 + 1, g1, sem1)
                wait_chunk(i0, g0, sem0)
                pltpu.sync_copy(g0, acc.at[dall.at[i0]], add=True)
                nxt = jnp.minimum(i0 + 2, HALF - 1)
                gather_chunk(nxt, g0, sem0)
                wait_chunk(i0 + 1, g1, sem1)
                pltpu.sync_copy(g1, acc.at[dall.at[i0 + 1]], add=True)
                return carry

            lax.fori_loop(0, HALF // 2, chunk2, 0)
            wait_chunk(0, g0, sem0)
        plsc.subcore_barrier()

        for kk in range(RPT // CHUNK):
            pltpu.sync_copy(acc.at[pl.ds(base + kk * CHUNK, CHUNK)], g1)
            pltpu.sync_copy(g1, out.at[g, cid, pl.ds(base + kk * CHUNK, CHUNK)])


def _sc_aggregate(h1, h2, s1, d1, s2, d2):
    k = pl.kernel(
        _agg_body,
        out_type=jax.ShapeDtypeStruct((2, NC, NP, D), jnp.float32),
        mesh=plsc.VectorSubcoreMesh(**_MESH),
        compiler_params=pltpu.CompilerParams(needs_layout_passes=False),
        scratch_types=[
            pltpu.VMEM((HALF, CHUNK), jnp.int32),
            pltpu.VMEM((HALF, CHUNK), jnp.int32),
            pltpu.VMEM((CHUNK, D), jnp.float32),
            pltpu.VMEM((CHUNK, D), jnp.float32),
            pltpu.VMEM_SHARED((NP, D), jnp.float32),
            pltpu.SemaphoreType.DMA,
            pltpu.SemaphoreType.DMA,
            pltpu.SemaphoreType.DMA,
            pltpu.SemaphoreType.DMA,
        ],
    )
    return k(h1, h2, s1, d1, s2, d2)


_BPT = BSEL // NW


def _gather_body(c1, c2, idx2d, out, idxb, gbuf, sem):
    cid = lax.axis_index("c")
    sid = lax.axis_index("s")
    wid = cid * NS + sid
    pltpu.sync_copy(idx2d.at[wid], idxb)
    pltpu.async_copy(c1.at[idxb], gbuf, sem).wait()
    pltpu.sync_copy(gbuf, out.at[0, pl.ds(wid * _BPT, _BPT)])
    pltpu.async_copy(c2.at[idxb], gbuf, sem).wait()
    pltpu.sync_copy(gbuf, out.at[1, pl.ds(wid * _BPT, _BPT)])


def _sc_gather(code1, code2, idx2d):
    k = pl.kernel(
        _gather_body,
        out_type=jax.ShapeDtypeStruct((2, BSEL, D), jnp.float32),
        mesh=plsc.VectorSubcoreMesh(**_MESH),
        compiler_params=pltpu.CompilerParams(needs_layout_passes=False),
        scratch_types=[
            pltpu.VMEM((_BPT,), jnp.int32),
            pltpu.VMEM((_BPT, D), jnp.float32),
            pltpu.SemaphoreType.DMA,
        ],
    )
    return k(code1, code2, idx2d)


_RB = 1024


def _norm_scale_body(dref, x1ref, x2ref, n1s, n1d, n2s, n2d, h1ref, h2ref):
    deg = jnp.sum(dref[...], axis=0)
    norm = lax.rsqrt(jnp.where(deg > 0, deg, 1.0))
    n1s[...] = norm[:, 0:1]
    n1d[...] = norm[:, 1:2]
    n2s[...] = norm[:, 2:3]
    n2d[...] = norm[:, 3:4]
    h1ref[...] = x1ref[...] * norm[:, 0:1]
    h2ref[...] = x2ref[...] * norm[:, 2:3]


def _tc_norm_scale(deg_parts, x1, x2):
    nspec = pl.BlockSpec((_RB, 1), lambda i: (i, 0))
    xspec = pl.BlockSpec((_RB, D), lambda i: (i, 0))
    return pl.pallas_call(
        _norm_scale_body,
        grid=(NP // _RB,),
        in_specs=[pl.BlockSpec((NW, _RB, 4), lambda i: (0, i, 0)), xspec, xspec],
        out_specs=[nspec, nspec, nspec, nspec, xspec, xspec],
        out_shape=[jax.ShapeDtypeStruct((NP, 1), jnp.float32)] * 4
        + [jax.ShapeDtypeStruct((NP, D), jnp.float32)] * 2,
    )(deg_parts, x1, x2)


def _finish_body(pref, wref, bref, ndref, nsref, oref, *, relu):
    p = pref[0, 0] + pref[0, 1]
    y = jnp.dot(p, wref[...], preferred_element_type=jnp.float32)
    y = y * ndref[...] + bref[...]
    if relu:
        y = jnp.maximum(y, 0.0) * nsref[...]
    oref[...] = y


def _tc_finish(parts, g, W, b, nd, ns, relu):
    return pl.pallas_call(
        functools.partial(_finish_body, relu=relu),
        grid=(NP // _RB,),
        in_specs=[
            pl.BlockSpec((1, NC, _RB, D), lambda i: (g, 0, i, 0)),
            pl.BlockSpec((D, D), lambda i: (0, 0)),
            pl.BlockSpec((1, D), lambda i: (0, 0)),
            pl.BlockSpec((_RB, 1), lambda i: (i, 0)),
            pl.BlockSpec((_RB, 1), lambda i: (i, 0)),
        ],
        out_specs=pl.BlockSpec((_RB, D), lambda i: (i, 0)),
        out_shape=jax.ShapeDtypeStruct((NP, D), jnp.float32),
    )(parts, W, b.reshape(1, D), nd, ns)


def _buildc_body(cref, oref):
    c1 = cref[0]
    c2 = cref[1]
    inv1 = 1.0 / jnp.maximum(jnp.sqrt(jnp.sum(c1 * c1, axis=1, keepdims=True)), 1e-12)
    inv2 = 1.0 / jnp.maximum(jnp.sqrt(jnp.sum(c2 * c2, axis=1, keepdims=True)), 1e-12)
    a = c1 * inv1
    b = c2 * inv2
    s = 1.0 / jnp.sqrt(6.0)
    oref[...] = jnp.concatenate([a * s, b * s, (a + b) * s], axis=1)


def _tc_buildc(cpair):
    return pl.pallas_call(
        _buildc_body,
        grid=(BSEL // _RB,),
        in_specs=[pl.BlockSpec((2, _RB, D), lambda i: (0, i, 0))],
        out_specs=pl.BlockSpec((_RB, 3 * D), lambda i: (i, 0)),
        out_shape=jax.ShapeDtypeStruct((BSEL, 3 * D), jnp.float32),
    )(cpair)


def _zmm_body(aref, bref, oref):
    i = pl.program_id(0)
    j = pl.program_id(1)

    @pl.when(j < i)
    def _():
        oref[...] = jnp.zeros_like(oref)

    @pl.when(j >= i)
    def _():
        z = lax.dot_general(aref[...], bref[...], (((1,), (1,)), ((), ())),
                            preferred_element_type=jnp.float32)
        rows = i * _RB + lax.broadcasted_iota(jnp.int32, (_RB, _RB), 0)
        cols = j * _RB + lax.broadcasted_iota(jnp.int32, (_RB, _RB), 1)
        oref[...] = jnp.where(cols >= rows, z, 0.0)


def _tc_zmatmul(cmat):
    return pl.pallas_call(
        _zmm_body,
        grid=(BSEL // _RB, BSEL // _RB),
        in_specs=[
            pl.BlockSpec((_RB, 3 * D), lambda i, j: (i, 0)),
            pl.BlockSpec((_RB, 3 * D), lambda i, j: (j, 0)),
        ],
        out_specs=pl.BlockSpec((_RB, _RB), lambda i, j: (i, j)),
        out_shape=jax.ShapeDtypeStruct((BSEL, BSEL), jnp.float32),
    )(cmat, cmat)


def kernel(raw1, edge_index1, raw2, edge_index2, index,
           W11, b11, W12, b12, W21, b21, W22, b22):
    s1 = _tile_edges(edge_index1[0])
    d1 = _tile_edges(edge_index1[1])
    s2 = _tile_edges(edge_index2[0])
    d2 = _tile_edges(edge_index2[1])
    x1 = jnp.pad(raw1, ((0, NP - N), (0, 0)))
    x2 = jnp.pad(raw2, ((0, NP - N), (0, 0)))
    idx2d = index.astype(jnp.int32).reshape(NW, BSEL // NW)

    deg_parts = _sc_degrees(s1, d1, s2, d2)
    n1s, n1d, n2s, n2d, h1, h2 = _tc_norm_scale(deg_parts, x1, x2)
    p1 = _sc_aggregate(h1, h2, s1, d1, s2, d2)
    h1b = _tc_finish(p1, 0, W11, b11, n1d, n1s, relu=True)
    h2b = _tc_finish(p1, 1, W21, b21, n2d, n2s, relu=True)
    p2 = _sc_aggregate(h1b, h2b, s1, d1, s2, d2)
    code1 = _tc_finish(p2, 0, W12, b12, n1d, n1d, relu=False)
    code2 = _tc_finish(p2, 1, W22, b22, n2d, n2d, relu=False)
    cpair = _sc_gather(code1, code2, idx2d)
    cmat = _tc_buildc(cpair)
    z = _tc_zmatmul(cmat)
    return (z, 0)


# speedup vs baseline: 1.2167x; 1.0024x over previous
import functools

import jax
import jax.numpy as jnp
from jax import lax
from jax.experimental import pallas as pl
from jax.experimental.pallas import tpu as pltpu
from jax.experimental.pallas import tpu_sc as plsc

N = 10000
NP = 10240
D = 128
BSEL = 4096
E = 320000

NC = 2
NS = 16
NW = NC * NS
CHUNK = 128
NCHUNK = 80
HALF = NCHUNK // 2
EPAD = NW * NCHUNK * CHUNK
PADIDX = N
RPT = NP // NS

_MESH = dict(core_axis_name="c", subcore_axis_name="s")


def _tile_edges(e):
    pad = jnp.full((EPAD - E,), PADIDX, jnp.int32)
    return jnp.concatenate([e.astype(jnp.int32), pad]).reshape(NW, NCHUNK, CHUNK)


def _deg_body(s1, d1, s2, d2, out, dacc, idxb):
    cid = lax.axis_index("c")
    sid = lax.axis_index("s")
    wid = cid * NS + sid

    def zero(i, carry):
        dacc[pl.ds(i * 16, 16)] = jnp.zeros((16,), jnp.float32)
        return carry

    lax.fori_loop(0, NP * 4 // 16, zero, 0)

    ones = jnp.full((16,), 1.0, jnp.float32)
    for a, arr in enumerate((s1, d1, s2, d2)):
        pltpu.sync_copy(arr.at[wid], idxb)

        def chunk(i, carry):
            for j in range(CHUNK // 16):
                v = idxb[i, pl.ds(j * 16, 16)]
                plsc.addupdate_scatter(dacc, [v * 4 + a], ones)
            return carry

        lax.fori_loop(0, NCHUNK, chunk, 0, unroll=2)

    pltpu.sync_copy(dacc, out.at[wid])


def _sc_degrees(s1, d1, s2, d2):
    k = pl.kernel(
        _deg_body,
        out_type=jax.ShapeDtypeStruct((NW, NP * 4), jnp.float32),
        mesh=plsc.VectorSubcoreMesh(**_MESH),
        compiler_params=pltpu.CompilerParams(needs_layout_passes=False),
        scratch_types=[
            pltpu.VMEM((NP * 4,), jnp.float32),
            pltpu.VMEM((NCHUNK, CHUNK), jnp.int32),
        ],
    )
    return k(s1, d1, s2, d2).reshape(NW, NP, 4)


def _agg_body(h1, h2, s1, d1, s2, d2, out,
              sall, dall, g0, g1, acc, sem0, sem1, sem2, sem3):
    cid = lax.axis_index("c")
    sid = lax.axis_index("s")
    wid = cid * NS + sid
    base = sid * RPT

    for g, (hg, sg, dg) in enumerate(((h1, s1, d1), (h2, s2, d2))):
        def zrow(r, carry):
            for j in range(D // 16):
                g0[r, pl.ds(j * 16, 16)] = jnp.zeros((16,), jnp.float32)
            return carry

        lax.fori_loop(0, CHUNK, zrow, 0)
        for kk in range(RPT // CHUNK):
            pltpu.sync_copy(g0, acc.at[pl.ds(base + kk * CHUNK, CHUNK)])
        plsc.subcore_barrier()

        def gather_chunk(i, gbuf, sem):
            pltpu.async_copy(hg.at[sall.at[i, pl.ds(0, 64)]],
                             gbuf.at[pl.ds(0, 64)], sem)
            pltpu.async_copy(hg.at[sall.at[i, pl.ds(64, 64)]],
                             gbuf.at[pl.ds(64, 64)], sem)

        def wait_chunk(i, gbuf, sem):
            pltpu.make_async_copy(hg.at[sall.at[i, pl.ds(0, 64)]],
                                  gbuf.at[pl.ds(0, 64)], sem).wait()
            pltpu.make_async_copy(hg.at[sall.at[i, pl.ds(64, 64)]],
                                  gbuf.at[pl.ds(64, 64)], sem).wait()

        for h in range(NCHUNK // HALF):
            pltpu.sync_copy(sg.at[wid, pl.ds(h * HALF, HALF)], sall)
            pltpu.sync_copy(dg.at[wid, pl.ds(h * HALF, HALF)], dall)
            gather_chunk(0, g0, sem0)

            def chunk2(k, carry):
                i0 = 2 * k
                gather_chunk(i0 + 1, g1, sem1)
                wait_chunk(i0, g0, sem0)
                pltpu.sync_copy(g0, acc.at[dall.at[i0]], add=True)
                nxt = jnp.minimum(i0 + 2, HALF - 1)
                gather_chunk(nxt, g0, sem0)
                wait_chunk(i0 + 1, g1, sem1)
                pltpu.sync_copy(g1, acc.at[dall.at[i0 + 1]], add=True)
                return carry

            lax.fori_loop(0, HALF // 2, chunk2, 0)
            wait_chunk(0, g0, sem0)
        plsc.subcore_barrier()

        for kk in range(RPT // CHUNK):
            pltpu.sync_copy(acc.at[pl.ds(base + kk * CHUNK, CHUNK)], g1)
            pltpu.sync_copy(g1, out.at[g, cid, pl.ds(base + kk * CHUNK, CHUNK)])


def _sc_aggregate(h1, h2, s1, d1, s2, d2):
    k = pl.kernel(
        _agg_body,
        out_type=jax.ShapeDtypeStruct((2, NC, NP, D), jnp.float32),
        mesh=plsc.VectorSubcoreMesh(**_MESH),
        compiler_params=pltpu.CompilerParams(needs_layout_passes=False),
        scratch_types=[
            pltpu.VMEM((HALF, CHUNK), jnp.int32),
            pltpu.VMEM((HALF, CHUNK), jnp.int32),
            pltpu.VMEM((CHUNK, D), jnp.float32),
            pltpu.VMEM((CHUNK, D), jnp.float32),
            pltpu.VMEM_SHARED((NP, D), jnp.float32),
            pltpu.SemaphoreType.DMA,
            pltpu.SemaphoreType.DMA,
            pltpu.SemaphoreType.DMA,
            pltpu.SemaphoreType.DMA,
        ],
    )
    return k(h1, h2, s1, d1, s2, d2)


_BPT = BSEL // NW


def _gather_body(c1, c2, idx2d, out, idxb, gbuf, sem):
    cid = lax.axis_index("c")
    sid = lax.axis_index("s")
    wid = cid * NS + sid
    pltpu.sync_copy(idx2d.at[wid], idxb)
    pltpu.async_copy(c1.at[idxb], gbuf, sem).wait()
    pltpu.sync_copy(gbuf, out.at[0, pl.ds(wid * _BPT, _BPT)])
    pltpu.async_copy(c2.at[idxb], gbuf, sem).wait()
    pltpu.sync_copy(gbuf, out.at[1, pl.ds(wid * _BPT, _BPT)])


def _sc_gather(code1, code2, idx2d):
    k = pl.kernel(
        _gather_body,
        out_type=jax.ShapeDtypeStruct((2, BSEL, D), jnp.float32),
        mesh=plsc.VectorSubcoreMesh(**_MESH),
        compiler_params=pltpu.CompilerParams(needs_layout_passes=False),
        scratch_types=[
            pltpu.VMEM((_BPT,), jnp.int32),
            pltpu.VMEM((_BPT, D), jnp.float32),
            pltpu.SemaphoreType.DMA,
        ],
    )
    return k(code1, code2, idx2d)


_RB = 1024


def _norm_scale_body(dref, x1ref, x2ref, n1s, n1d, n2s, n2d, h1ref, h2ref):
    deg = jnp.sum(dref[...], axis=0)
    norm = lax.rsqrt(jnp.where(deg > 0, deg, 1.0))
    n1s[...] = norm[:, 0:1]
    n1d[...] = norm[:, 1:2]
    n2s[...] = norm[:, 2:3]
    n2d[...] = norm[:, 3:4]
    h1ref[...] = x1ref[...] * norm[:, 0:1]
    h2ref[...] = x2ref[...] * norm[:, 2:3]


def _tc_norm_scale(deg_parts, x1, x2):
    nspec = pl.BlockSpec((_RB, 1), lambda i: (i, 0))
    xspec = pl.BlockSpec((_RB, D), lambda i: (i, 0))
    return pl.pallas_call(
        _norm_scale_body,
        grid=(NP // _RB,),
        in_specs=[pl.BlockSpec((NW, _RB, 4), lambda i: (0, i, 0)), xspec, xspec],
        out_specs=[nspec, nspec, nspec, nspec, xspec, xspec],
        out_shape=[jax.ShapeDtypeStruct((NP, 1), jnp.float32)] * 4
        + [jax.ShapeDtypeStruct((NP, D), jnp.float32)] * 2,
    )(deg_parts, x1, x2)


def _finish_body(pref, wref, bref, ndref, nsref, oref, *, relu):
    p = pref[0, 0] + pref[0, 1]
    y = jnp.dot(p, wref[...], preferred_element_type=jnp.float32)
    y = y * ndref[...] + bref[...]
    if relu:
        y = jnp.maximum(y, 0.0) * nsref[...]
    oref[...] = y


def _tc_finish(parts, g, W, b, nd, ns, relu):
    return pl.pallas_call(
        functools.partial(_finish_body, relu=relu),
        grid=(NP // _RB,),
        in_specs=[
            pl.BlockSpec((1, NC, _RB, D), lambda i: (g, 0, i, 0)),
            pl.BlockSpec((D, D), lambda i: (0, 0)),
            pl.BlockSpec((1, D), lambda i: (0, 0)),
            pl.BlockSpec((_RB, 1), lambda i: (i, 0)),
            pl.BlockSpec((_RB, 1), lambda i: (i, 0)),
        ],
        out_specs=pl.BlockSpec((_RB, D), lambda i: (i, 0)),
        out_shape=jax.ShapeDtypeStruct((NP, D), jnp.float32),
    )(parts, W, b.reshape(1, D), nd, ns)


def _buildc_body(cref, oref):
    c1 = cref[0]
    c2 = cref[1]
    inv1 = 1.0 / jnp.maximum(jnp.sqrt(jnp.sum(c1 * c1, axis=1, keepdims=True)), 1e-12)
    inv2 = 1.0 / jnp.maximum(jnp.sqrt(jnp.sum(c2 * c2, axis=1, keepdims=True)), 1e-12)
    a = c1 * inv1
    b = c2 * inv2
    s = 1.0 / jnp.sqrt(6.0)
    oref[...] = jnp.concatenate([a * s, b * s, (a + b) * s],
                                axis=1).astype(jnp.bfloat16)


def _tc_buildc(cpair):
    return pl.pallas_call(
        _buildc_body,
        grid=(BSEL // _RB,),
        in_specs=[pl.BlockSpec((2, _RB, D), lambda i: (0, i, 0))],
        out_specs=pl.BlockSpec((_RB, 3 * D), lambda i: (i, 0)),
        out_shape=jax.ShapeDtypeStruct((BSEL, 3 * D), jnp.bfloat16),
    )(cpair)


def _zmm_body(aref, bref, oref):
    i = pl.program_id(0)
    j = pl.program_id(1)

    @pl.when(j < i)
    def _():
        oref[...] = jnp.zeros_like(oref)

    @pl.when(j >= i)
    def _():
        z = lax.dot_general(aref[...], bref[...], (((1,), (1,)), ((), ())),
                            preferred_element_type=jnp.float32)
        rows = i * _RB + lax.broadcasted_iota(jnp.int32, (_RB, _RB), 0)
        cols = j * _RB + lax.broadcasted_iota(jnp.int32, (_RB, _RB), 1)
        oref[...] = jnp.where(cols >= rows, z, 0.0)


def _tc_zmatmul(cmat):
    return pl.pallas_call(
        _zmm_body,
        grid=(BSEL // _RB, BSEL // _RB),
        in_specs=[
            pl.BlockSpec((_RB, 3 * D), lambda i, j: (i, 0)),
            pl.BlockSpec((_RB, 3 * D), lambda i, j: (j, 0)),
        ],
        out_specs=pl.BlockSpec((_RB, _RB), lambda i, j: (i, j)),
        out_shape=jax.ShapeDtypeStruct((BSEL, BSEL), jnp.float32),
    )(cmat, cmat)


def kernel(raw1, edge_index1, raw2, edge_index2, index,
           W11, b11, W12, b12, W21, b21, W22, b22):
    s1 = _tile_edges(edge_index1[0])
    d1 = _tile_edges(edge_index1[1])
    s2 = _tile_edges(edge_index2[0])
    d2 = _tile_edges(edge_index2[1])
    x1 = jnp.pad(raw1, ((0, NP - N), (0, 0)))
    x2 = jnp.pad(raw2, ((0, NP - N), (0, 0)))
    idx2d = index.astype(jnp.int32).reshape(NW, BSEL // NW)

    deg_parts = _sc_degrees(s1, d1, s2, d2)
    n1s, n1d, n2s, n2d, h1, h2 = _tc_norm_scale(deg_parts, x1, x2)
    p1 = _sc_aggregate(h1, h2, s1, d1, s2, d2)
    h1b = _tc_finish(p1, 0, W11, b11, n1d, n1s, relu=True)
    h2b = _tc_finish(p1, 1, W21, b21, n2d, n2s, relu=True)
    p2 = _sc_aggregate(h1b, h2b, s1, d1, s2, d2)
    code1 = _tc_finish(p2, 0, W12, b12, n1d, n1d, relu=False)
    code2 = _tc_finish(p2, 1, W22, b22, n2d, n2d, relu=False)
    cpair = _sc_gather(code1, code2, idx2d)
    cmat = _tc_buildc(cpair)
    z = _tc_zmatmul(cmat)
    return (z, 0)
